# baseline (speedup 1.0000x reference)
import jax
import jax.numpy as jnp
from jax.experimental import pallas as pl
from jax.experimental.pallas import tpu as pltpu

TOPK = 8
NE = 64
NMIN = 8
BT = 1024


def _body(minid_ref, x_ref, out_ref, acc_ref):
    i = pl.program_id(0)
    n = pl.num_programs(0)

    @pl.when(i == 0)
    def _init():
        for m in range(NMIN):
            acc_ref[m] = 0

    v = x_ref[...]
    lane = jax.lax.broadcasted_iota(jnp.int32, v.shape, 1)
    for m in range(NMIN):
        e = minid_ref[m]
        col = jnp.max(jnp.where(lane == e, v, -jnp.inf), axis=1, keepdims=True)
        gt = jnp.sum((v > col).astype(jnp.int32), axis=1, keepdims=True)
        tie = jnp.sum(((v == col) & (lane < e)).astype(jnp.int32), axis=1,
                      keepdims=True)
        present = jnp.max((gt + tie < TOPK).astype(jnp.int32))
        acc_ref[m] = jnp.maximum(acc_ref[m], present)

    @pl.when(i == n - 1)
    def _fin():
        matched = jnp.int32(0)
        for m in range(NMIN):
            uniq = jnp.int32(1)
            for mp in range(m):
                uniq = uniq * (minid_ref[mp] != minid_ref[m]).astype(jnp.int32)
            matched = matched + uniq * acc_ref[m]
        total = float(TOPK * n * BT)
        out_ref[0] = 1.0 - matched.astype(jnp.float32) / total


def kernel(log_probs, min_k_expert_indices):
    x = log_probs.reshape(-1, NE)
    minid = min_k_expert_indices.astype(jnp.int32)
    grid = x.shape[0] // BT
    out = pl.pallas_call(
        _body,
        grid=(grid,),
        in_specs=[
            pl.BlockSpec(memory_space=pltpu.SMEM),
            pl.BlockSpec((BT, NE), lambda i: (i, 0)),
        ],
        out_specs=pl.BlockSpec(memory_space=pltpu.SMEM),
        out_shape=jax.ShapeDtypeStruct((1,), jnp.float32),
        scratch_shapes=[pltpu.SMEM((NMIN,), jnp.int32)],
    )(minid, x)
    return out[0]


# speedup vs baseline: 3.3987x; 3.3987x over previous
import functools

import jax
import jax.numpy as jnp
from jax import lax
from jax.experimental import pallas as pl
from jax.experimental.pallas import tpu as pltpu
from jax.experimental.pallas import tpu_sc as plsc

TOPK = 8
NE = 64
NMIN = 8
L = 16
NV = NE // L
NSUB = 16
NTOK = 4 * 8192
TPW = NTOK // NSUB
CHUNK = 256
NCHUNK = TPW // CHUNK


def _dg(v, idx):
    return v.at[idx].get(mode="promise_in_bounds")


def _sc_body(x_hbm, minid_hbm, out_hbm, chunk_v, minid_v, pres_v, all_v,
             stage_v, outstage_v, shared):
    sid = lax.axis_index("s")
    lane = lax.broadcasted_iota(jnp.int32, (L,), 0)
    lanes_g = [lane + L * k for k in range(NV)]
    fzero = jnp.zeros((L,), jnp.float32)
    fone = fzero + 1.0

    rots = []
    for sh in (8, 4, 2, 1):
        idx = lane + sh
        rots.append(jnp.where(idx >= L, idx - L, idx))

    pltpu.sync_copy(minid_hbm, minid_v)
    mvec = minid_v[...]
    escal = [mvec[m] for m in range(NMIN)]

    pres_v[...] = jnp.zeros((NMIN * L,), jnp.float32)
    all_v[...] = fzero

    base = sid * TPW

    def tok_body(t, carry):
        a0 = all_v[...]

        @pl.when(a0[0] == 0.0)
        def _():
            tbase = t * NE
            rows = [chunk_v[pl.ds(tbase + L * k, L)] for k in range(NV)]
            slots = []
            for m in range(NMIN):
                off = escal[m]
                vrow = chunk_v[pl.ds(tbase + (off // L) * L, L)]
                ve = _dg(vrow, jnp.zeros((L,), jnp.int32) + (off % L))
                cv = fzero
                for k in range(NV):
                    row = rows[k]
                    ltf = jnp.where(lanes_g[k] < off, fone, fzero)
                    cv = (cv + jnp.where(row > ve, fone, fzero)
                          + jnp.where(row == ve, ltf, fzero))
                for r in rots:
                    cv = cv + _dg(cv, r)
                pres = jnp.where(cv < float(TOPK), fone, fzero)
                s = jnp.maximum(pres_v[pl.ds(L * m, L)], pres)
                pres_v[pl.ds(L * m, L)] = s
                slots.append(s)
            acc = slots[0]
            for s in slots[1:]:
                acc = jnp.minimum(acc, s)
            all_v[...] = acc

        return carry

    def chunk_body(ci, carry):
        a0 = all_v[...]

        @pl.when(a0[0] == 0.0)
        def _():
            pltpu.sync_copy(x_hbm.at[pl.ds((base + ci * CHUNK) * NE,
                                           CHUNK * NE)], chunk_v)
            lax.fori_loop(0, CHUNK, tok_body, jnp.int32(0))

        return carry

    lax.fori_loop(0, NCHUNK, chunk_body, jnp.int32(0))

    pltpu.sync_copy(pres_v, shared.at[sid])
    plsc.subcore_barrier()

    @pl.when(sid == 0)
    def _finalize():
        gf = [fzero] * NMIN
        for w in range(NSUB):
            pltpu.sync_copy(shared.at[w], stage_v)
            for m in range(NMIN):
                gf[m] = jnp.maximum(gf[m], stage_v[pl.ds(L * m, L)])
        matched = fzero
        for m in range(NMIN):
            uniq = fone
            for mp in range(m):
                uniq = uniq * jnp.where(escal[mp] != escal[m], fone, fzero)
            matched = matched + uniq * gf[m]
        loss = 1.0 - matched / float(TOPK * NTOK)
        outstage_v[...] = loss
        pltpu.sync_copy(outstage_v, out_hbm)


@jax.jit
def _run(x, minid16):
    mesh = plsc.VectorSubcoreMesh(core_axis_name="c", subcore_axis_name="s",
                                  num_cores=1)
    f = functools.partial(
        pl.kernel,
        out_type=jax.ShapeDtypeStruct((L,), jnp.float32),
        mesh=mesh,
        scratch_types=[
            pltpu.VMEM((CHUNK * NE,), jnp.float32),
            pltpu.VMEM((L,), jnp.int32),
            pltpu.VMEM((NMIN * L,), jnp.float32),
            pltpu.VMEM((L,), jnp.float32),
            pltpu.VMEM((NMIN * L,), jnp.float32),
            pltpu.VMEM((L,), jnp.float32),
            pltpu.VMEM_SHARED((NSUB, NMIN * L), jnp.float32),
        ],
    )(_sc_body)
    return f(x, minid16)


def kernel(log_probs, min_k_expert_indices):
    x = log_probs.reshape(NTOK * NE)
    minid = min_k_expert_indices.astype(jnp.int32)
    minid16 = jnp.concatenate([minid, jnp.zeros((L - NMIN,), jnp.int32)])
    out = _run(x, minid16)
    return out[0]
